# baseline (speedup 1.0000x reference)
import functools

import jax
import jax.numpy as jnp
from jax import lax
from jax.experimental import pallas as pl
from jax.experimental.pallas import tpu as pltpu
from jax.experimental.pallas import tpu_sc as plsc

_L = 16
_NB = 1024
_CH = 8000


def _tc1_body(x_ref, u_ref, v_ref, b_ref, xT_ref, a_ref, bm_ref):
    xT = x_ref[...].T
    xT_ref[...] = xT
    a_ref[...] = jnp.dot(u_ref[...], xT, preferred_element_type=jnp.float32) + b_ref[...]
    bm_ref[...] = jnp.dot(v_ref[...], xT, preferred_element_type=jnp.float32)


def _lrelu_gate(s, a):
    z = a + s
    h = jnp.where(z >= 0, z, 0.2 * z)
    return jnp.where(s == -jnp.inf, 0.0, h)


def _tc2_body(xT_ref, s_ref, a0_ref, u1x_ref, u1h_ref, v1x_ref, v1h_ref,
              b_ref, h0_ref, a1_ref, b1_ref):
    h0 = _lrelu_gate(s_ref[...], a0_ref[...])
    h0_ref[...] = h0
    xT = xT_ref[...]
    dot = lambda w, m: jnp.dot(w, m, preferred_element_type=jnp.float32)
    a1_ref[...] = dot(u1x_ref[...], xT) + dot(u1h_ref[...], h0) + b_ref[...]
    b1_ref[...] = dot(v1x_ref[...], xT) + dot(v1h_ref[...], h0)


def _tc3_body(xT_ref, h0_ref, s_ref, a1_ref, wx_ref, wh0_ref, wh1_ref,
              b_ref, out_ref):
    h1 = _lrelu_gate(s_ref[...], a1_ref[...])
    xT = xT_ref[...]
    dot = lambda w, m: jnp.dot(w, m, preferred_element_type=jnp.float32)
    resT = (dot(wx_ref[...], xT) + dot(wh0_ref[...], h0_ref[...])
            + dot(wh1_ref[...], h1) + b_ref[...] + xT)
    out_ref[...] = resT.T


def _make_tc_calls(N, C, interpret=False):
    g = N // _NB
    full = pl.BlockSpec((C, C), lambda i: (0, 0))
    bias = pl.BlockSpec((C, 1), lambda i: (0, 0))
    colT = pl.BlockSpec((C, _NB), lambda i: (0, i))
    rows = pl.BlockSpec((_NB, C), lambda i: (i, 0))
    fTN = jax.ShapeDtypeStruct((C, N), jnp.float32)

    tc1 = pl.pallas_call(
        _tc1_body, grid=(g,),
        in_specs=[rows, full, full, bias],
        out_specs=[colT, colT, colT],
        out_shape=[fTN, fTN, fTN],
        interpret=interpret)
    tc2 = pl.pallas_call(
        _tc2_body, grid=(g,),
        in_specs=[colT, colT, colT, full, full, full, full, bias],
        out_specs=[colT, colT, colT],
        out_shape=[fTN, fTN, fTN],
        interpret=interpret)
    tc3 = pl.pallas_call(
        _tc3_body, grid=(g,),
        in_specs=[colT, colT, colT, colT, full, full, full, bias],
        out_specs=rows,
        out_shape=jax.ShapeDtypeStruct((N, C), jnp.float32),
        interpret=interpret)
    return tc1, tc2, tc3


def _make_segmax(N, C, E):
    info = plsc.get_sparse_core_info()
    NC, NS = info.num_cores, info.num_subcores
    NW = NC * NS
    assert C % NW == 0
    FPW = C // NW
    PW = FPW * N
    assert PW % _L == 0 and PW % 8 == 0
    assert E % _CH == 0 and _CH % _L == 0
    mesh = plsc.VectorSubcoreMesh(core_axis_name="c", subcore_axis_name="s")

    @functools.partial(
        pl.kernel, mesh=mesh,
        out_type=jax.ShapeDtypeStruct((C * N,), jnp.float32),
        compiler_params=pltpu.CompilerParams(needs_layout_passes=False),
        scratch_types=[
            pltpu.VMEM((PW,), jnp.float32),
            pltpu.VMEM((PW,), jnp.float32),
            pltpu.VMEM((_CH,), jnp.int32),
            pltpu.VMEM((_CH,), jnp.int32),
        ])
    def segmax(bT_hbm, src_hbm, dst_hbm, out_hbm, bT_v, s_v, src_v, dst_v):
        wid = lax.axis_index("s") * NC + lax.axis_index("c")
        fbase = wid * PW
        pltpu.sync_copy(bT_hbm.at[pl.ds(fbase, PW)], bT_v)

        neg = jnp.full((_L,), -jnp.inf, jnp.float32)

        def init_body(i, c):
            s_v[pl.ds(i * _L, _L)] = neg
            return c
        lax.fori_loop(0, PW // _L, init_body, 0)

        def vec_body(vi, c):
            s_idx = src_v[pl.ds(vi * _L, _L)]
            d_idx = dst_v[pl.ds(vi * _L, _L)]
            vals = [plsc.load_gather(bT_v, [s_idx + j * N]) for j in range(FPW)]
            djs = [d_idx + j * N for j in range(FPW)]
            cnt, _ = plsc.scan_count(d_idx)
            hasdup = jnp.any(cnt > 0)

            def no_dup(_):
                curs = [plsc.load_gather(s_v, [djs[j]]) for j in range(FPW)]
                for j in range(FPW):
                    plsc.store_scatter(s_v, [djs[j]], vals[j],
                                       mask=vals[j] > curs[j])
                return 0

            def with_dup(_):
                def fix_body(_):
                    rem = jnp.bool_(False)
                    for j in range(FPW):
                        cur = plsc.load_gather(s_v, [djs[j]])
                        need = vals[j] > cur
                        plsc.store_scatter(s_v, [djs[j]], vals[j], mask=need)
                        rem = jnp.logical_or(rem, jnp.any(need))
                    return rem
                lax.while_loop(lambda r: r, fix_body, jnp.bool_(True))
                return 0

            lax.cond(hasdup, with_dup, no_dup, 0)
            return c

        def chunk_body(ci, c):
            pltpu.sync_copy(src_hbm.at[pl.ds(ci * _CH, _CH)], src_v)
            pltpu.sync_copy(dst_hbm.at[pl.ds(ci * _CH, _CH)], dst_v)
            lax.fori_loop(0, _CH // _L, vec_body, 0)
            return c
        lax.fori_loop(0, E // _CH, chunk_body, 0)

        pltpu.sync_copy(s_v, out_hbm.at[pl.ds(fbase, PW)])

    return segmax


def kernel(x, edge_index, W0, b0, W1, b1, Wd, bd):
    Nin, C = x.shape
    E = edge_index.shape[1]
    src = edge_index[0]
    dst = edge_index[1]
    N = ((Nin + _NB - 1) // _NB) * _NB
    if N != Nin:
        x = jnp.pad(x, ((0, N - Nin), (0, 0)))
    if E % _CH:
        pad = _CH - E % _CH
        src = jnp.concatenate([src, jnp.broadcast_to(src[-1:], (pad,))])
        dst = jnp.concatenate([dst, jnp.broadcast_to(dst[-1:], (pad,))])
        E += pad

    U0t = (W0[:C] - W0[C:]).T
    V0t = W0[C:].T
    U1 = W1[:2 * C] - W1[2 * C:]
    V1 = W1[2 * C:]
    U1xt, U1ht = U1[:C].T, U1[C:].T
    V1xt, V1ht = V1[:C].T, V1[C:].T
    Wxt, Wh0t, Wh1t = Wd[:C].T, Wd[C:2 * C].T, Wd[2 * C:].T
    b0c = b0.reshape(C, 1)
    b1c = b1.reshape(C, 1)
    bdc = bd.reshape(C, 1)

    tc1, tc2, tc3 = _make_tc_calls(N, C)
    segmax = _make_segmax(N, C, E)

    xT, A0T, B0T = tc1(x, U0t, V0t, b0c)
    S0T = segmax(B0T.reshape(-1), src, dst).reshape(C, N)
    h0T, A1T, B1T = tc2(xT, S0T, A0T, U1xt, U1ht, V1xt, V1ht, b1c)
    S1T = segmax(B1T.reshape(-1), src, dst).reshape(C, N)
    out = tc3(xT, h0T, S1T, A1T, Wxt, Wh0t, Wh1t, bdc)
    return out[:Nin]


# speedup vs baseline: 1.4799x; 1.4799x over previous
import functools

import jax
import jax.numpy as jnp
from jax import lax
from jax.experimental import pallas as pl
from jax.experimental.pallas import tpu as pltpu
from jax.experimental.pallas import tpu_sc as plsc

_L = 16
_NB = 1024
_CH = 8000


def _tc1_body(x_ref, u_ref, v_ref, b_ref, xT_ref, a_ref, bm_ref):
    xT = x_ref[...].T
    xT_ref[...] = xT
    a_ref[...] = jnp.dot(u_ref[...], xT, preferred_element_type=jnp.float32) + b_ref[...]
    bm_ref[...] = jnp.dot(v_ref[...], xT, preferred_element_type=jnp.float32)


def _lrelu_gate(s, a):
    z = a + s
    h = jnp.where(z >= 0, z, 0.2 * z)
    return jnp.where(s == -jnp.inf, 0.0, h)


def _tc2_body(xT_ref, s_ref, a0_ref, u1x_ref, u1h_ref, v1x_ref, v1h_ref,
              b_ref, h0_ref, a1_ref, b1_ref):
    h0 = _lrelu_gate(s_ref[...], a0_ref[...])
    h0_ref[...] = h0
    xT = xT_ref[...]
    dot = lambda w, m: jnp.dot(w, m, preferred_element_type=jnp.float32)
    a1_ref[...] = dot(u1x_ref[...], xT) + dot(u1h_ref[...], h0) + b_ref[...]
    b1_ref[...] = dot(v1x_ref[...], xT) + dot(v1h_ref[...], h0)


def _tc3_body(xT_ref, h0_ref, s_ref, a1_ref, wx_ref, wh0_ref, wh1_ref,
              b_ref, out_ref):
    h1 = _lrelu_gate(s_ref[...], a1_ref[...])
    xT = xT_ref[...]
    dot = lambda w, m: jnp.dot(w, m, preferred_element_type=jnp.float32)
    resT = (dot(wx_ref[...], xT) + dot(wh0_ref[...], h0_ref[...])
            + dot(wh1_ref[...], h1) + b_ref[...] + xT)
    out_ref[...] = resT.T


def _make_tc_calls(N, C, interpret=False):
    g = N // _NB
    full = pl.BlockSpec((C, C), lambda i: (0, 0))
    bias = pl.BlockSpec((C, 1), lambda i: (0, 0))
    colT = pl.BlockSpec((C, _NB), lambda i: (0, i))
    rows = pl.BlockSpec((_NB, C), lambda i: (i, 0))
    fTN = jax.ShapeDtypeStruct((C, N), jnp.float32)

    tc1 = pl.pallas_call(
        _tc1_body, grid=(g,),
        in_specs=[rows, full, full, bias],
        out_specs=[colT, colT, colT],
        out_shape=[fTN, fTN, fTN],
        interpret=interpret)
    tc2 = pl.pallas_call(
        _tc2_body, grid=(g,),
        in_specs=[colT, colT, colT, full, full, full, full, bias],
        out_specs=[colT, colT, colT],
        out_shape=[fTN, fTN, fTN],
        interpret=interpret)
    tc3 = pl.pallas_call(
        _tc3_body, grid=(g,),
        in_specs=[colT, colT, colT, colT, full, full, full, bias],
        out_specs=rows,
        out_shape=jax.ShapeDtypeStruct((N, C), jnp.float32),
        interpret=interpret)
    return tc1, tc2, tc3


def _make_segmax(N, C, E):
    info = plsc.get_sparse_core_info()
    NC, NS = info.num_cores, info.num_subcores
    NW = NC * NS
    assert C % NW == 0
    FPW = C // NW
    PW = FPW * N
    assert PW % _L == 0 and PW % 8 == 0
    assert E % _CH == 0 and _CH % _L == 0
    mesh = plsc.VectorSubcoreMesh(core_axis_name="c", subcore_axis_name="s")

    @functools.partial(
        pl.kernel, mesh=mesh,
        out_type=jax.ShapeDtypeStruct((C * N,), jnp.float32),
        compiler_params=pltpu.CompilerParams(needs_layout_passes=False),
        scratch_types=[
            pltpu.VMEM((PW,), jnp.float32),
            pltpu.VMEM((PW,), jnp.float32),
            pltpu.VMEM((_CH,), jnp.int32),
            pltpu.VMEM((_CH,), jnp.int32),
        ])
    def segmax(bT_hbm, src_hbm, dst_hbm, out_hbm, bT_v, s_v, src_v, dst_v):
        wid = lax.axis_index("s") * NC + lax.axis_index("c")
        fbase = wid * PW
        pltpu.sync_copy(bT_hbm.at[pl.ds(fbase, PW)], bT_v)

        neg = jnp.full((_L,), -jnp.inf, jnp.float32)

        def init_body(i, c):
            s_v[pl.ds(i * _L, _L)] = neg
            return c
        lax.fori_loop(0, PW // _L, init_body, 0)

        def vec_body(vi, c):
            s_idx = src_v[pl.ds(vi * _L, _L)]
            d_idx = dst_v[pl.ds(vi * _L, _L)]
            vals = [plsc.load_gather(bT_v, [s_idx + j * N]) for j in range(FPW)]
            djs = [d_idx + j * N for j in range(FPW)]
            cnt, _ = plsc.scan_count(d_idx)
            hasdup = jnp.any(cnt > 0)

            def fix_body(_):
                curs = [plsc.load_gather(s_v, [djs[j]]) for j in range(FPW)]
                needs = [vals[j] > curs[j] for j in range(FPW)]
                for j in range(FPW):
                    plsc.store_scatter(s_v, [djs[j]], vals[j], mask=needs[j])
                anyv = needs[0]
                for j in range(1, FPW):
                    anyv = jnp.logical_or(anyv, needs[j])
                return jnp.logical_and(hasdup, jnp.any(anyv))
            lax.while_loop(lambda r: r, fix_body, jnp.bool_(True))
            return c

        def chunk_body(ci, c):
            pltpu.sync_copy(src_hbm.at[pl.ds(ci * _CH, _CH)], src_v)
            pltpu.sync_copy(dst_hbm.at[pl.ds(ci * _CH, _CH)], dst_v)
            lax.fori_loop(0, _CH // _L, vec_body, 0)
            return c
        lax.fori_loop(0, E // _CH, chunk_body, 0)

        pltpu.sync_copy(s_v, out_hbm.at[pl.ds(fbase, PW)])

    return segmax


def kernel(x, edge_index, W0, b0, W1, b1, Wd, bd):
    Nin, C = x.shape
    E = edge_index.shape[1]
    src = edge_index[0]
    dst = edge_index[1]
    N = ((Nin + _NB - 1) // _NB) * _NB
    if N != Nin:
        x = jnp.pad(x, ((0, N - Nin), (0, 0)))
    if E % _CH:
        pad = _CH - E % _CH
        src = jnp.concatenate([src, jnp.broadcast_to(src[-1:], (pad,))])
        dst = jnp.concatenate([dst, jnp.broadcast_to(dst[-1:], (pad,))])
        E += pad

    U0t = (W0[:C] - W0[C:]).T
    V0t = W0[C:].T
    U1 = W1[:2 * C] - W1[2 * C:]
    V1 = W1[2 * C:]
    U1xt, U1ht = U1[:C].T, U1[C:].T
    V1xt, V1ht = V1[:C].T, V1[C:].T
    Wxt, Wh0t, Wh1t = Wd[:C].T, Wd[C:2 * C].T, Wd[2 * C:].T
    b0c = b0.reshape(C, 1)
    b1c = b1.reshape(C, 1)
    bdc = bd.reshape(C, 1)

    tc1, tc2, tc3 = _make_tc_calls(N, C)
    segmax = _make_segmax(N, C, E)

    xT, A0T, B0T = tc1(x, U0t, V0t, b0c)
    S0T = segmax(B0T.reshape(-1), src, dst).reshape(C, N)
    h0T, A1T, B1T = tc2(xT, S0T, A0T, U1xt, U1ht, V1xt, V1ht, b1c)
    S1T = segmax(B1T.reshape(-1), src, dst).reshape(C, N)
    out = tc3(xT, h0T, S1T, A1T, Wxt, Wh0t, Wh1t, bdc)
    return out[:Nin]


# speedup vs baseline: 1.5552x; 1.0509x over previous
import functools

import jax
import jax.numpy as jnp
from jax import lax
from jax.experimental import pallas as pl
from jax.experimental.pallas import tpu as pltpu
from jax.experimental.pallas import tpu_sc as plsc

_L = 16
_NB = 1024
_CH = 8000
_G = 4


def _tc1_body(x_ref, u_ref, v_ref, b_ref, xT_ref, a_ref, bm_ref):
    xT = x_ref[...].T
    xT_ref[...] = xT
    a_ref[...] = jnp.dot(u_ref[...], xT, preferred_element_type=jnp.float32) + b_ref[...]
    bm_ref[...] = jnp.dot(v_ref[...], xT, preferred_element_type=jnp.float32)


def _lrelu_gate(s, a):
    z = a + s
    h = jnp.where(z >= 0, z, 0.2 * z)
    return jnp.where(s == -jnp.inf, 0.0, h)


def _tc2_body(xT_ref, s_ref, a0_ref, u1x_ref, u1h_ref, v1x_ref, v1h_ref,
              b_ref, h0_ref, a1_ref, b1_ref):
    h0 = _lrelu_gate(s_ref[...], a0_ref[...])
    h0_ref[...] = h0
    xT = xT_ref[...]
    dot = lambda w, m: jnp.dot(w, m, preferred_element_type=jnp.float32)
    a1_ref[...] = dot(u1x_ref[...], xT) + dot(u1h_ref[...], h0) + b_ref[...]
    b1_ref[...] = dot(v1x_ref[...], xT) + dot(v1h_ref[...], h0)


def _tc3_body(xT_ref, h0_ref, s_ref, a1_ref, wx_ref, wh0_ref, wh1_ref,
              b_ref, out_ref):
    h1 = _lrelu_gate(s_ref[...], a1_ref[...])
    xT = xT_ref[...]
    dot = lambda w, m: jnp.dot(w, m, preferred_element_type=jnp.float32)
    resT = (dot(wx_ref[...], xT) + dot(wh0_ref[...], h0_ref[...])
            + dot(wh1_ref[...], h1) + b_ref[...] + xT)
    out_ref[...] = resT.T


def _make_tc_calls(N, C, interpret=False):
    g = N // _NB
    full = pl.BlockSpec((C, C), lambda i: (0, 0))
    bias = pl.BlockSpec((C, 1), lambda i: (0, 0))
    colT = pl.BlockSpec((C, _NB), lambda i: (0, i))
    rows = pl.BlockSpec((_NB, C), lambda i: (i, 0))
    fTN = jax.ShapeDtypeStruct((C, N), jnp.float32)

    tc1 = pl.pallas_call(
        _tc1_body, grid=(g,),
        in_specs=[rows, full, full, bias],
        out_specs=[colT, colT, colT],
        out_shape=[fTN, fTN, fTN],
        interpret=interpret)
    tc2 = pl.pallas_call(
        _tc2_body, grid=(g,),
        in_specs=[colT, colT, colT, full, full, full, full, bias],
        out_specs=[colT, colT, colT],
        out_shape=[fTN, fTN, fTN],
        interpret=interpret)
    tc3 = pl.pallas_call(
        _tc3_body, grid=(g,),
        in_specs=[colT, colT, colT, colT, full, full, full, bias],
        out_specs=rows,
        out_shape=jax.ShapeDtypeStruct((N, C), jnp.float32),
        interpret=interpret)
    return tc1, tc2, tc3


def _make_segmax(N, C, E):
    info = plsc.get_sparse_core_info()
    NC, NS = info.num_cores, info.num_subcores
    NW = NC * NS
    assert C % NW == 0
    FPW = C // NW
    PW = FPW * N
    assert PW % _L == 0 and PW % 8 == 0
    assert E % _CH == 0 and _CH % _L == 0
    mesh = plsc.VectorSubcoreMesh(core_axis_name="c", subcore_axis_name="s")

    @functools.partial(
        pl.kernel, mesh=mesh,
        out_type=jax.ShapeDtypeStruct((C * N,), jnp.float32),
        compiler_params=pltpu.CompilerParams(needs_layout_passes=False),
        scratch_types=[
            pltpu.VMEM((PW,), jnp.float32),
            pltpu.VMEM((PW,), jnp.float32),
            pltpu.VMEM((_CH,), jnp.int32),
            pltpu.VMEM((_CH,), jnp.int32),
        ])
    def segmax(bT_hbm, src_hbm, dst_hbm, out_hbm, bT_v, s_v, src_v, dst_v):
        wid = lax.axis_index("s") * NC + lax.axis_index("c")
        fbase = wid * PW
        pltpu.sync_copy(bT_hbm.at[pl.ds(fbase, PW)], bT_v)

        neg = jnp.full((_L,), -jnp.inf, jnp.float32)

        def init_body(i, c):
            s_v[pl.ds(i * _L, _L)] = neg
            return c
        lax.fori_loop(0, PW // _L, init_body, 0)

        def onepass(off):
            s_idx = src_v[pl.ds(off, _L)]
            d_idx = dst_v[pl.ds(off, _L)]
            vals = [plsc.load_gather(bT_v, [s_idx + j * N]) for j in range(FPW)]
            djs = [d_idx + j * N for j in range(FPW)]
            cnt, last = plsc.scan_count(d_idx)
            dup = jnp.logical_or(jnp.logical_not(last), cnt > 0)
            curs = [plsc.load_gather(s_v, [djs[j]]) for j in range(FPW)]
            needs = [vals[j] > curs[j] for j in range(FPW)]
            for j in range(FPW):
                plsc.store_scatter(s_v, [djs[j]], vals[j], mask=needs[j])
            anyv = needs[0]
            for j in range(1, FPW):
                anyv = jnp.logical_or(anyv, needs[j])
            return jnp.logical_and(dup, anyv)

        def fixpoint(off):
            s_idx = src_v[pl.ds(off, _L)]
            d_idx = dst_v[pl.ds(off, _L)]
            vals = [plsc.load_gather(bT_v, [s_idx + j * N]) for j in range(FPW)]
            djs = [d_idx + j * N for j in range(FPW)]
            cnt, _ = plsc.scan_count(d_idx)
            hasdup = jnp.any(cnt > 0)

            def fix_body(_):
                curs = [plsc.load_gather(s_v, [djs[j]]) for j in range(FPW)]
                needs = [vals[j] > curs[j] for j in range(FPW)]
                for j in range(FPW):
                    plsc.store_scatter(s_v, [djs[j]], vals[j], mask=needs[j])
                anyv = needs[0]
                for j in range(1, FPW):
                    anyv = jnp.logical_or(anyv, needs[j])
                return jnp.logical_and(hasdup, jnp.any(anyv))
            lax.while_loop(lambda r: r, fix_body, jnp.bool_(True))

        def group_body(gi, c):
            base = gi * (_G * _L)
            conflict = onepass(base)
            for u in range(1, _G):
                conflict = jnp.logical_or(conflict, onepass(base + u * _L))

            @pl.when(jnp.any(conflict))
            def _():
                for u in range(_G):
                    fixpoint(base + u * _L)
            return c

        def chunk_body(ci, c):
            pltpu.sync_copy(src_hbm.at[pl.ds(ci * _CH, _CH)], src_v)
            pltpu.sync_copy(dst_hbm.at[pl.ds(ci * _CH, _CH)], dst_v)
            lax.fori_loop(0, _CH // (_G * _L), group_body, 0)
            return c
        lax.fori_loop(0, E // _CH, chunk_body, 0)

        pltpu.sync_copy(s_v, out_hbm.at[pl.ds(fbase, PW)])

    return segmax


def kernel(x, edge_index, W0, b0, W1, b1, Wd, bd):
    Nin, C = x.shape
    E = edge_index.shape[1]
    src = edge_index[0]
    dst = edge_index[1]
    N = ((Nin + _NB - 1) // _NB) * _NB
    if N != Nin:
        x = jnp.pad(x, ((0, N - Nin), (0, 0)))
    if E % _CH:
        pad = _CH - E % _CH
        src = jnp.concatenate([src, jnp.broadcast_to(src[-1:], (pad,))])
        dst = jnp.concatenate([dst, jnp.broadcast_to(dst[-1:], (pad,))])
        E += pad

    U0t = (W0[:C] - W0[C:]).T
    V0t = W0[C:].T
    U1 = W1[:2 * C] - W1[2 * C:]
    V1 = W1[2 * C:]
    U1xt, U1ht = U1[:C].T, U1[C:].T
    V1xt, V1ht = V1[:C].T, V1[C:].T
    Wxt, Wh0t, Wh1t = Wd[:C].T, Wd[C:2 * C].T, Wd[2 * C:].T
    b0c = b0.reshape(C, 1)
    b1c = b1.reshape(C, 1)
    bdc = bd.reshape(C, 1)

    tc1, tc2, tc3 = _make_tc_calls(N, C)
    segmax = _make_segmax(N, C, E)

    xT, A0T, B0T = tc1(x, U0t, V0t, b0c)
    S0T = segmax(B0T.reshape(-1), src, dst).reshape(C, N)
    h0T, A1T, B1T = tc2(xT, S0T, A0T, U1xt, U1ht, V1xt, V1ht, b1c)
    S1T = segmax(B1T.reshape(-1), src, dst).reshape(C, N)
    out = tc3(xT, h0T, S1T, A1T, Wxt, Wh0t, Wh1t, bdc)
    return out[:Nin]
